# Initial kernel scaffold; baseline (speedup 1.0000x reference)
import jax
import jax.numpy as jnp
from jax.experimental import pallas as pl


def kernel(x, edge_index, W0, b0, W1, b1, W2, b2, W3, b3, W4, b4):
    raise NotImplementedError("write your pallas kernel here")


# speedup vs baseline: 3.4230x; 3.4230x over previous
import functools

import jax
import jax.numpy as jnp
from jax import lax
from jax.experimental import pallas as pl
from jax.experimental.pallas import tpu as pltpu
from jax.experimental.pallas import tpu_sc as plsc

N = 10000
NP = 10240
E = 320000
EP = 327680
ER = 2560
RPT = NP // 16
NC = 2
NS = 16
W = 64
BN = 512


@functools.cache
def _mesh():
    return plsc.VectorSubcoreMesh(
        core_axis_name="c", subcore_axis_name="s",
        num_cores=NC, num_subcores=NS)


def _deg_kernel(src_hbm, dst_hbm, ones_hbm, zeros_hbm, out_hbm,
                idx_v, ones_v, deg_sh):
    c = lax.axis_index("c")
    s = lax.axis_index("s")
    cpt = ER // NS

    @pl.when(c == 0)
    def _():
        pltpu.sync_copy(src_hbm.at[pl.ds(s * cpt, cpt)], idx_v)

    @pl.when(c == 1)
    def _():
        pltpu.sync_copy(dst_hbm.at[pl.ds(s * cpt, cpt)], idx_v)

    pltpu.sync_copy(ones_hbm, ones_v)
    pltpu.sync_copy(zeros_hbm, deg_sh.at[pl.ds(s * RPT, RPT)])
    plsc.subcore_barrier()

    def body(g, carry):
        pltpu.sync_copy(ones_v, deg_sh.at[idx_v.at[g]], add=True)
        return carry

    lax.fori_loop(0, cpt, body, 0)
    plsc.subcore_barrier()
    pltpu.sync_copy(deg_sh.at[pl.ds(s * RPT, RPT)],
                    out_hbm.at[c, pl.ds(s * RPT, RPT)])


def _make_deg():
    return functools.partial(
        pl.kernel,
        out_type=jax.ShapeDtypeStruct((NC, NP, 16), jnp.float32),
        mesh=_mesh(),
        compiler_params=pltpu.CompilerParams(use_tc_tiling_on_sc=False),
        scratch_types=[
            pltpu.VMEM((ER // NS, 128), jnp.int32),
            pltpu.VMEM((128, 16), jnp.float32),
            pltpu.VMEM_SHARED((NP, 16), jnp.float32),
        ],
    )(_deg_kernel)


def _agg_body(nplanes, g_hbm, src_hbm, dst_hbm, zeros_hbm, out_hbm,
              src_v, dst_v, rows0, rows1, agg_sh, sem0, sem1):
    c = lax.axis_index("c")
    s = lax.axis_index("s")
    if nplanes >= 2:
        cpt = ER // NS
        base = s * cpt
    else:
        cpt = ER // (NC * NS)
        base = (c * NS + s) * cpt
    pltpu.sync_copy(src_hbm.at[pl.ds(base, cpt)], src_v)
    pltpu.sync_copy(dst_hbm.at[pl.ds(base, cpt)], dst_v)

    def one_plane(table, out_slot):
        pltpu.sync_copy(zeros_hbm, agg_sh.at[pl.ds(s * RPT, RPT)])
        plsc.subcore_barrier()

        def body(g, carry):
            cp0 = pltpu.async_copy(table.at[src_v.at[2 * g]], rows0, sem0)
            cp1 = pltpu.async_copy(table.at[src_v.at[2 * g + 1]], rows1, sem1)
            cp0.wait()
            pltpu.sync_copy(rows0, agg_sh.at[dst_v.at[2 * g]], add=True)
            cp1.wait()
            pltpu.sync_copy(rows1, agg_sh.at[dst_v.at[2 * g + 1]], add=True)
            return carry

        lax.fori_loop(0, cpt // 2, body, 0)
        plsc.subcore_barrier()
        pltpu.sync_copy(agg_sh.at[pl.ds(s * RPT, RPT)],
                        out_hbm.at[out_slot, pl.ds(s * RPT, RPT)])
        plsc.subcore_barrier()

    if nplanes == 1:
        one_plane(g_hbm, c)
    else:
        for pp in range(nplanes // 2):
            plane = c * (nplanes // 2) + pp
            one_plane(g_hbm.at[plane], plane)


def _make_agg(nplanes):
    cpt = ER // NS if nplanes >= 2 else ER // (NC * NS)
    return functools.partial(
        pl.kernel,
        out_type=jax.ShapeDtypeStruct(
            (nplanes if nplanes >= 2 else 2, NP, W), jnp.float32),
        mesh=_mesh(),
        compiler_params=pltpu.CompilerParams(use_tc_tiling_on_sc=False),
        scratch_types=[
            pltpu.VMEM((cpt, 128), jnp.int32),
            pltpu.VMEM((cpt, 128), jnp.int32),
            pltpu.VMEM((128, W), jnp.float32),
            pltpu.VMEM((128, W), jnp.float32),
            pltpu.VMEM_SHARED((NP, W), jnp.float32),
            pltpu.SemaphoreType.DMA,
            pltpu.SemaphoreType.DMA,
        ],
    )(functools.partial(_agg_body, nplanes))


def _norm_body(deg_ref, x_ref, s_ref, d_ref, g0_ref):
    s = lax.rsqrt(jnp.maximum(deg_ref[0, :, 0:1], 1.0))
    d = lax.rsqrt(jnp.maximum(deg_ref[1, :, 0:1], 1.0))
    s_ref[...] = s
    d_ref[...] = d
    xs = x_ref[...] * s
    g0_ref[0] = xs[:, 0:W]
    g0_ref[1] = xs[:, W:2 * W]


def _norm_call(deg, x_pad):
    nb = NP // BN
    return pl.pallas_call(
        _norm_body,
        grid=(nb,),
        in_specs=[
            pl.BlockSpec((NC, BN, 16), lambda i: (0, i, 0)),
            pl.BlockSpec((BN, 128), lambda i: (i, 0)),
        ],
        out_specs=(
            pl.BlockSpec((BN, 1), lambda i: (i, 0)),
            pl.BlockSpec((BN, 1), lambda i: (i, 0)),
            pl.BlockSpec((2, BN, W), lambda i: (0, i, 0)),
        ),
        out_shape=(
            jax.ShapeDtypeStruct((NP, 1), jnp.float32),
            jax.ShapeDtypeStruct((NP, 1), jnp.float32),
            jax.ShapeDtypeStruct((2, NP, W), jnp.float32),
        ),
    )(deg, x_pad)


def _matmul_planes(a_ref, w_ref, np_in):
    acc = jnp.dot(a_ref[0], w_ref[0:W, :], preferred_element_type=jnp.float32)
    for p in range(1, np_in):
        acc = acc + jnp.dot(a_ref[p], w_ref[p * W:(p + 1) * W, :],
                            preferred_element_type=jnp.float32)
    return acc


def _layer_body(np_in, partial_sum, a_ref, w_ref, b_ref, s_ref, d_ref,
                out_ref):
    if partial_sum:
        a0 = a_ref[0] + a_ref[1]
        acc = jnp.dot(a0, w_ref[0:W, :], preferred_element_type=jnp.float32)
    else:
        acc = _matmul_planes(a_ref, w_ref, np_in)
    h = s_ref[...] * jnp.maximum(d_ref[...] * acc + b_ref[...], 0.0)
    out_ref[0] = h[:, 0:W]
    out_ref[1] = h[:, W:2 * W]


def _layer_call(np_in, partial_sum, a, w, b, sv, dv):
    nb = NP // BN
    kdim = w.shape[0]
    return pl.pallas_call(
        functools.partial(_layer_body, np_in, partial_sum),
        grid=(nb, 2),
        in_specs=[
            pl.BlockSpec((a.shape[0], BN, W), lambda i, q: (0, i, 0)),
            pl.BlockSpec((kdim, 128), lambda i, q: (0, q)),
            pl.BlockSpec((1, 128), lambda i, q: (0, q)),
            pl.BlockSpec((BN, 1), lambda i, q: (i, 0)),
            pl.BlockSpec((BN, 1), lambda i, q: (i, 0)),
        ],
        out_specs=pl.BlockSpec((2, BN, W), lambda i, q: (q, i, 0)),
        out_shape=jax.ShapeDtypeStruct((4, NP, W), jnp.float32),
    )(a, w, b, sv, dv)


def _tail_body(a_ref, w3_ref, b3_ref, w4_ref, s_ref, d_ref, z_ref):
    sv = s_ref[...]
    dv = d_ref[...]
    w3 = w3_ref[...]
    z = None
    for q in (0, 1):
        acc = None
        for p in range(4):
            t = jnp.dot(a_ref[p], w3[p * W:(p + 1) * W, q * 128:(q + 1) * 128],
                        preferred_element_type=jnp.float32)
            acc = t if acc is None else acc + t
        h = sv * jnp.maximum(dv * acc + b3_ref[0:1, q * 128:(q + 1) * 128],
                             0.0)
        zq = jnp.dot(h, w4_ref[q * 128:(q + 1) * 128, :],
                     preferred_element_type=jnp.float32)
        z = zq if z is None else z + zq
    z_ref[...] = z


def _tail_call(a, w3, b3, w4p, sv, dv):
    nb = NP // BN
    return pl.pallas_call(
        _tail_body,
        grid=(nb,),
        in_specs=[
            pl.BlockSpec((4, BN, W), lambda i: (0, i, 0)),
            pl.BlockSpec((256, 256), lambda i: (0, 0)),
            pl.BlockSpec((1, 256), lambda i: (0, 0)),
            pl.BlockSpec((256, 64), lambda i: (0, 0)),
            pl.BlockSpec((BN, 1), lambda i: (i, 0)),
            pl.BlockSpec((BN, 1), lambda i: (i, 0)),
        ],
        out_specs=pl.BlockSpec((BN, 64), lambda i: (i, 0)),
        out_shape=jax.ShapeDtypeStruct((NP, 64), jnp.float32),
    )(a, w3, b3, w4p, sv, dv)


def _final_body(z_ref, d_ref, b4_ref, out_ref):
    out_ref[...] = d_ref[...] * (z_ref[0] + z_ref[1]) + b4_ref[...]


def _final_call(z, dv, b4p):
    nb = NP // BN
    return pl.pallas_call(
        _final_body,
        grid=(nb,),
        in_specs=[
            pl.BlockSpec((NC, BN, 64), lambda i: (0, i, 0)),
            pl.BlockSpec((BN, 1), lambda i: (i, 0)),
            pl.BlockSpec((1, 64), lambda i: (0, 0)),
        ],
        out_specs=pl.BlockSpec((BN, 64), lambda i: (i, 0)),
        out_shape=jax.ShapeDtypeStruct((NP, 64), jnp.float32),
    )(z, dv, b4p)


def kernel(x, edge_index, W0, b0, W1, b1, W2, b2, W3, b3, W4, b4):
    src = edge_index[0].astype(jnp.int32)
    dst = edge_index[1].astype(jnp.int32)
    pad = EP - E
    src_g = jnp.concatenate([src, jnp.zeros((pad,), jnp.int32)]).reshape(ER, 128)
    src_d = jnp.concatenate([src, jnp.full((pad,), N, jnp.int32)]).reshape(ER, 128)
    dst_p = jnp.concatenate([dst, jnp.full((pad,), N, jnp.int32)]).reshape(ER, 128)

    x_pad = jnp.pad(x, ((0, NP - N), (0, 0)))
    b0r = b0.reshape(1, 256)
    b1r = b1.reshape(1, 256)
    b2r = b2.reshape(1, 256)
    b3r = b3.reshape(1, 256)
    w4p = jnp.pad(W4, ((0, 0), (0, 64 - W4.shape[1])))
    b4p = jnp.pad(b4, (0, 64 - b4.shape[0])).reshape(1, 64)

    ones16 = jnp.ones((128, 16), jnp.float32)
    zeros16 = jnp.zeros((RPT, 16), jnp.float32)
    zerosW = jnp.zeros((RPT, W), jnp.float32)

    deg = _make_deg()(src_d, dst_p, ones16, zeros16)
    sv, dv, g0 = _norm_call(deg, x_pad)

    a0 = _make_agg(2)(g0, src_g, dst_p, zerosW)
    g1 = _layer_call(2, False, a0, W0, b0r, sv, dv)
    a1 = _make_agg(4)(g1, src_g, dst_p, zerosW)
    g2 = _layer_call(4, False, a1, W1, b1r, sv, dv)
    a2 = _make_agg(4)(g2, src_g, dst_p, zerosW)
    g3 = _layer_call(4, False, a2, W2, b2r, sv, dv)
    a3 = _make_agg(4)(g3, src_g, dst_p, zerosW)
    z = _tail_call(a3, W3, b3r, w4p, sv, dv)
    za = _make_agg(1)(z, src_g, dst_p, zerosW)
    out = _final_call(za, dv, b4p)
    return out[:N, :40]


# speedup vs baseline: 3.9815x; 1.1632x over previous
import functools

import jax
import jax.numpy as jnp
from jax import lax
from jax.experimental import pallas as pl
from jax.experimental.pallas import tpu as pltpu
from jax.experimental.pallas import tpu_sc as plsc

N = 10000
NP = 10240
E = 320000
EP = 327680
ER = 2560
RPT = NP // 16
NC = 2
NS = 16
W = 64
BN = 512


@functools.cache
def _mesh():
    return plsc.VectorSubcoreMesh(
        core_axis_name="c", subcore_axis_name="s",
        num_cores=NC, num_subcores=NS)


def _deg_kernel(src_hbm, dst_hbm, ones_hbm, zeros_hbm, out_hbm,
                idx_v, ones_v, deg_sh, ds0, ds1, ds2, ds3):
    dsem = (ds0, ds1, ds2, ds3)
    c = lax.axis_index("c")
    s = lax.axis_index("s")
    cpt = ER // NS

    @pl.when(c == 0)
    def _():
        pltpu.sync_copy(src_hbm.at[pl.ds(s * cpt, cpt)], idx_v)

    @pl.when(c == 1)
    def _():
        pltpu.sync_copy(dst_hbm.at[pl.ds(s * cpt, cpt)], idx_v)

    pltpu.sync_copy(ones_hbm, ones_v)
    pltpu.sync_copy(zeros_hbm, deg_sh.at[pl.ds(s * RPT, RPT)])
    plsc.subcore_barrier()

    def body(g, carry):
        cps = [pltpu.async_copy(ones_v, deg_sh.at[idx_v.at[4 * g + b]],
                                dsem[b], add=True) for b in range(4)]
        for cp in cps:
            cp.wait()
        return carry

    lax.fori_loop(0, cpt // 4, body, 0)
    plsc.subcore_barrier()
    pltpu.sync_copy(deg_sh.at[pl.ds(s * RPT, RPT)],
                    out_hbm.at[c, pl.ds(s * RPT, RPT)])


def _make_deg():
    return functools.partial(
        pl.kernel,
        out_type=jax.ShapeDtypeStruct((NC, NP, 16), jnp.float32),
        mesh=_mesh(),
        compiler_params=pltpu.CompilerParams(use_tc_tiling_on_sc=False),
        scratch_types=[
            pltpu.VMEM((ER // NS, 128), jnp.int32),
            pltpu.VMEM((128, 16), jnp.float32),
            pltpu.VMEM_SHARED((NP, 16), jnp.float32),
            pltpu.SemaphoreType.DMA,
            pltpu.SemaphoreType.DMA,
            pltpu.SemaphoreType.DMA,
            pltpu.SemaphoreType.DMA,
        ],
    )(_deg_kernel)


def _agg_body(nplanes, g_hbm, src_hbm, dst_hbm, zeros_hbm, out_hbm,
              src_v, dst_v, r0, r1, r2, r3, agg_sh,
              gs0, gs1, gs2, gs3, ss0, ss1, ss2, ss3):
    rows = (r0, r1, r2, r3)
    gsem = (gs0, gs1, gs2, gs3)
    ssem = (ss0, ss1, ss2, ss3)
    c = lax.axis_index("c")
    s = lax.axis_index("s")
    if nplanes >= 2:
        cpt = ER // NS
        base = s * cpt
    else:
        cpt = ER // (NC * NS)
        base = (c * NS + s) * cpt
    pltpu.sync_copy(src_hbm.at[pl.ds(base, cpt)], src_v)
    pltpu.sync_copy(dst_hbm.at[pl.ds(base, cpt)], dst_v)

    def one_plane(table, out_slot):
        pltpu.sync_copy(zeros_hbm, agg_sh.at[pl.ds(s * RPT, RPT)])
        plsc.subcore_barrier()

        def gather(ch, b):
            return pltpu.async_copy(table.at[src_v.at[ch]], rows[b], gsem[b])

        def gather_wait(ch, b):
            pltpu.make_async_copy(table.at[src_v.at[ch]], rows[b],
                                  gsem[b]).wait()

        def scatter(ch, b):
            return pltpu.async_copy(rows[b], agg_sh.at[dst_v.at[ch]],
                                    ssem[b], add=True)

        for b in range(4):
            gather(b, b)

        def body(g, carry):
            cps = []
            for b in range(4):
                gather_wait(4 * g + b, b)
                cps.append(scatter(4 * g + b, b))
            for b in range(4):
                cps[b].wait()
                gather(4 * g + 4 + b, b)
            return carry

        lax.fori_loop(0, cpt // 4 - 1, body, 0)
        cps = []
        for b in range(4):
            ch = cpt - 4 + b
            gather_wait(ch, b)
            cps.append(scatter(ch, b))
        for cp in cps:
            cp.wait()
        plsc.subcore_barrier()
        pltpu.sync_copy(agg_sh.at[pl.ds(s * RPT, RPT)],
                        out_hbm.at[out_slot, pl.ds(s * RPT, RPT)])
        plsc.subcore_barrier()

    if nplanes == 1:
        one_plane(g_hbm, c)
    else:
        for pp in range(nplanes // 2):
            plane = c * (nplanes // 2) + pp
            one_plane(g_hbm.at[plane], plane)


def _make_agg(nplanes):
    cpt = ER // NS if nplanes >= 2 else ER // (NC * NS)
    return functools.partial(
        pl.kernel,
        out_type=jax.ShapeDtypeStruct(
            (nplanes if nplanes >= 2 else 2, NP, W), jnp.float32),
        mesh=_mesh(),
        compiler_params=pltpu.CompilerParams(use_tc_tiling_on_sc=False),
        scratch_types=(
            [pltpu.VMEM((cpt, 128), jnp.int32),
             pltpu.VMEM((cpt, 128), jnp.int32)] +
            [pltpu.VMEM((128, W), jnp.float32) for _ in range(4)] +
            [pltpu.VMEM_SHARED((NP, W), jnp.float32)] +
            [pltpu.SemaphoreType.DMA for _ in range(8)]
        ),
    )(functools.partial(_agg_body, nplanes))


def _norm_body(deg_ref, x_ref, s_ref, d_ref, g0_ref):
    s = lax.rsqrt(jnp.maximum(deg_ref[0, :, 0:1], 1.0))
    d = lax.rsqrt(jnp.maximum(deg_ref[1, :, 0:1], 1.0))
    s_ref[...] = s
    d_ref[...] = d
    xs = x_ref[...] * s
    g0_ref[0] = xs[:, 0:W]
    g0_ref[1] = xs[:, W:2 * W]


def _norm_call(deg, x_pad):
    nb = NP // BN
    return pl.pallas_call(
        _norm_body,
        grid=(nb,),
        in_specs=[
            pl.BlockSpec((NC, BN, 16), lambda i: (0, i, 0)),
            pl.BlockSpec((BN, 128), lambda i: (i, 0)),
        ],
        out_specs=(
            pl.BlockSpec((BN, 1), lambda i: (i, 0)),
            pl.BlockSpec((BN, 1), lambda i: (i, 0)),
            pl.BlockSpec((2, BN, W), lambda i: (0, i, 0)),
        ),
        out_shape=(
            jax.ShapeDtypeStruct((NP, 1), jnp.float32),
            jax.ShapeDtypeStruct((NP, 1), jnp.float32),
            jax.ShapeDtypeStruct((2, NP, W), jnp.float32),
        ),
    )(deg, x_pad)


def _matmul_planes(a_ref, w_ref, np_in):
    acc = jnp.dot(a_ref[0], w_ref[0:W, :], preferred_element_type=jnp.float32)
    for p in range(1, np_in):
        acc = acc + jnp.dot(a_ref[p], w_ref[p * W:(p + 1) * W, :],
                            preferred_element_type=jnp.float32)
    return acc


def _layer_body(np_in, partial_sum, a_ref, w_ref, b_ref, s_ref, d_ref,
                out_ref):
    if partial_sum:
        a0 = a_ref[0] + a_ref[1]
        acc = jnp.dot(a0, w_ref[0:W, :], preferred_element_type=jnp.float32)
    else:
        acc = _matmul_planes(a_ref, w_ref, np_in)
    h = s_ref[...] * jnp.maximum(d_ref[...] * acc + b_ref[...], 0.0)
    out_ref[0] = h[:, 0:W]
    out_ref[1] = h[:, W:2 * W]


def _layer_call(np_in, partial_sum, a, w, b, sv, dv):
    nb = NP // BN
    kdim = w.shape[0]
    return pl.pallas_call(
        functools.partial(_layer_body, np_in, partial_sum),
        grid=(nb, 2),
        in_specs=[
            pl.BlockSpec((a.shape[0], BN, W), lambda i, q: (0, i, 0)),
            pl.BlockSpec((kdim, 128), lambda i, q: (0, q)),
            pl.BlockSpec((1, 128), lambda i, q: (0, q)),
            pl.BlockSpec((BN, 1), lambda i, q: (i, 0)),
            pl.BlockSpec((BN, 1), lambda i, q: (i, 0)),
        ],
        out_specs=pl.BlockSpec((2, BN, W), lambda i, q: (q, i, 0)),
        out_shape=jax.ShapeDtypeStruct((4, NP, W), jnp.float32),
    )(a, w, b, sv, dv)


def _tail_body(a_ref, w3_ref, b3_ref, w4_ref, s_ref, d_ref, z_ref):
    sv = s_ref[...]
    dv = d_ref[...]
    w3 = w3_ref[...]
    z = None
    for q in (0, 1):
        acc = None
        for p in range(4):
            t = jnp.dot(a_ref[p], w3[p * W:(p + 1) * W, q * 128:(q + 1) * 128],
                        preferred_element_type=jnp.float32)
            acc = t if acc is None else acc + t
        h = sv * jnp.maximum(dv * acc + b3_ref[0:1, q * 128:(q + 1) * 128],
                             0.0)
        zq = jnp.dot(h, w4_ref[q * 128:(q + 1) * 128, :],
                     preferred_element_type=jnp.float32)
        z = zq if z is None else z + zq
    z_ref[...] = z


def _tail_call(a, w3, b3, w4p, sv, dv):
    nb = NP // BN
    return pl.pallas_call(
        _tail_body,
        grid=(nb,),
        in_specs=[
            pl.BlockSpec((4, BN, W), lambda i: (0, i, 0)),
            pl.BlockSpec((256, 256), lambda i: (0, 0)),
            pl.BlockSpec((1, 256), lambda i: (0, 0)),
            pl.BlockSpec((256, 64), lambda i: (0, 0)),
            pl.BlockSpec((BN, 1), lambda i: (i, 0)),
            pl.BlockSpec((BN, 1), lambda i: (i, 0)),
        ],
        out_specs=pl.BlockSpec((BN, 64), lambda i: (i, 0)),
        out_shape=jax.ShapeDtypeStruct((NP, 64), jnp.float32),
    )(a, w3, b3, w4p, sv, dv)


def _final_body(z_ref, d_ref, b4_ref, out_ref):
    out_ref[...] = d_ref[...] * (z_ref[0] + z_ref[1]) + b4_ref[...]


def _final_call(z, dv, b4p):
    nb = NP // BN
    return pl.pallas_call(
        _final_body,
        grid=(nb,),
        in_specs=[
            pl.BlockSpec((NC, BN, 64), lambda i: (0, i, 0)),
            pl.BlockSpec((BN, 1), lambda i: (i, 0)),
            pl.BlockSpec((1, 64), lambda i: (0, 0)),
        ],
        out_specs=pl.BlockSpec((BN, 64), lambda i: (i, 0)),
        out_shape=jax.ShapeDtypeStruct((NP, 64), jnp.float32),
    )(z, dv, b4p)


def kernel(x, edge_index, W0, b0, W1, b1, W2, b2, W3, b3, W4, b4):
    src = edge_index[0].astype(jnp.int32)
    dst = edge_index[1].astype(jnp.int32)
    pad = EP - E
    src_g = jnp.concatenate([src, jnp.zeros((pad,), jnp.int32)]).reshape(ER, 128)
    src_d = jnp.concatenate([src, jnp.full((pad,), N, jnp.int32)]).reshape(ER, 128)
    dst_p = jnp.concatenate([dst, jnp.full((pad,), N, jnp.int32)]).reshape(ER, 128)

    x_pad = jnp.pad(x, ((0, NP - N), (0, 0)))
    b0r = b0.reshape(1, 256)
    b1r = b1.reshape(1, 256)
    b2r = b2.reshape(1, 256)
    b3r = b3.reshape(1, 256)
    w4p = jnp.pad(W4, ((0, 0), (0, 64 - W4.shape[1])))
    b4p = jnp.pad(b4, (0, 64 - b4.shape[0])).reshape(1, 64)

    ones16 = jnp.ones((128, 16), jnp.float32)
    zeros16 = jnp.zeros((RPT, 16), jnp.float32)
    zerosW = jnp.zeros((RPT, W), jnp.float32)

    deg = _make_deg()(src_d, dst_p, ones16, zeros16)
    sv, dv, g0 = _norm_call(deg, x_pad)

    a0 = _make_agg(2)(g0, src_g, dst_p, zerosW)
    g1 = _layer_call(2, False, a0, W0, b0r, sv, dv)
    a1 = _make_agg(4)(g1, src_g, dst_p, zerosW)
    g2 = _layer_call(4, False, a1, W1, b1r, sv, dv)
    a2 = _make_agg(4)(g2, src_g, dst_p, zerosW)
    g3 = _layer_call(4, False, a2, W2, b2r, sv, dv)
    a3 = _make_agg(4)(g3, src_g, dst_p, zerosW)
    z = _tail_call(a3, W3, b3r, w4p, sv, dv)
    za = _make_agg(1)(z, src_g, dst_p, zerosW)
    out = _final_call(za, dv, b4p)
    return out[:N, :40]


# speedup vs baseline: 9.0555x; 2.2744x over previous
import functools

import jax
import jax.numpy as jnp
from jax import lax
from jax.experimental import pallas as pl
from jax.experimental.pallas import tpu as pltpu
from jax.experimental.pallas import tpu_sc as plsc

N = 10000
NP = 10240
E = 320000
EP = 327680
ER = 2560
RPT = NP // 16
NC = 2
NS = 16
W = 64
BN = 512


@functools.cache
def _mesh():
    return plsc.VectorSubcoreMesh(
        core_axis_name="c", subcore_axis_name="s",
        num_cores=NC, num_subcores=NS)


def _deg_kernel(src_hbm, dst_hbm, ones_hbm, zeros_hbm, out_hbm,
                idx_v, ones_v, deg_sh, ds0, ds1, ds2, ds3):
    dsem = (ds0, ds1, ds2, ds3)
    c = lax.axis_index("c")
    s = lax.axis_index("s")
    cpt = ER // NS

    @pl.when(c == 0)
    def _():
        pltpu.sync_copy(src_hbm.at[pl.ds(s * cpt, cpt)], idx_v)

    @pl.when(c == 1)
    def _():
        pltpu.sync_copy(dst_hbm.at[pl.ds(s * cpt, cpt)], idx_v)

    pltpu.sync_copy(ones_hbm, ones_v)
    pltpu.sync_copy(zeros_hbm, deg_sh.at[pl.ds(s * RPT, RPT)])
    plsc.subcore_barrier()

    def body(g, carry):
        cps = [pltpu.async_copy(ones_v, deg_sh.at[idx_v.at[4 * g + b]],
                                dsem[b], add=True) for b in range(4)]
        for cp in cps:
            cp.wait()
        return carry

    lax.fori_loop(0, cpt // 4, body, 0)
    plsc.subcore_barrier()
    pltpu.sync_copy(deg_sh.at[pl.ds(s * RPT, RPT)],
                    out_hbm.at[c, pl.ds(s * RPT, RPT)])


def _make_deg():
    return functools.partial(
        pl.kernel,
        out_type=jax.ShapeDtypeStruct((NC, NP, 16), jnp.float32),
        mesh=_mesh(),
        compiler_params=pltpu.CompilerParams(use_tc_tiling_on_sc=False),
        scratch_types=[
            pltpu.VMEM((ER // NS, 128), jnp.int32),
            pltpu.VMEM((128, 16), jnp.float32),
            pltpu.VMEM_SHARED((NP, 16), jnp.float32),
            pltpu.SemaphoreType.DMA,
            pltpu.SemaphoreType.DMA,
            pltpu.SemaphoreType.DMA,
            pltpu.SemaphoreType.DMA,
        ],
    )(_deg_kernel)


def _agg_body(nplanes, g_hbm, src_hbm, dst_hbm, zeros_hbm, out_hbm,
              src_v, dst_v, r0, r1, r2, r3, agg_sh,
              gs0, gs1, gs2, gs3, ss0, ss1, ss2, ss3):
    rows = (r0, r1, r2, r3)
    gsem = (gs0, gs1, gs2, gs3)
    ssem = (ss0, ss1, ss2, ss3)
    c = lax.axis_index("c")
    s = lax.axis_index("s")
    if nplanes >= 2:
        cpt = ER // NS
        base = s * cpt
    else:
        cpt = ER // (NC * NS)
        base = (c * NS + s) * cpt
    pltpu.sync_copy(src_hbm.at[pl.ds(base, cpt)], src_v)
    pltpu.sync_copy(dst_hbm.at[pl.ds(base, cpt)], dst_v)

    def one_plane(table, out_slot):
        pltpu.sync_copy(zeros_hbm, agg_sh.at[pl.ds(s * RPT, RPT)])
        plsc.subcore_barrier()

        def gather(ch, b):
            return pltpu.async_copy(table.at[src_v.at[ch]], rows[b], gsem[b])

        def gather_wait(ch, b):
            pltpu.make_async_copy(table.at[src_v.at[ch]], rows[b],
                                  gsem[b]).wait()

        def scatter(ch, b):
            return pltpu.async_copy(rows[b], agg_sh.at[dst_v.at[ch]],
                                    ssem[b], add=True)

        for b in range(4):
            gather(b, b)

        def body(g, carry):
            cps = []
            for b in range(4):
                gather_wait(4 * g + b, b)
                cps.append(scatter(4 * g + b, b))
            for b in range(4):
                cps[b].wait()
                gather(4 * g + 4 + b, b)
            return carry

        lax.fori_loop(0, cpt // 4 - 1, body, 0)
        cps = []
        for b in range(4):
            ch = cpt - 4 + b
            gather_wait(ch, b)
            cps.append(scatter(ch, b))
        for cp in cps:
            cp.wait()
        plsc.subcore_barrier()
        pltpu.sync_copy(agg_sh.at[pl.ds(s * RPT, RPT)],
                        out_hbm.at[out_slot, pl.ds(s * RPT, RPT)])
        plsc.subcore_barrier()

    if nplanes == 1:
        one_plane(g_hbm, c)
    else:
        for pp in range(nplanes // 2):
            plane = c * (nplanes // 2) + pp
            one_plane(g_hbm.at[plane], plane)


def _make_agg(nplanes):
    cpt = ER // NS if nplanes >= 2 else ER // (NC * NS)
    return functools.partial(
        pl.kernel,
        out_type=jax.ShapeDtypeStruct(
            (nplanes if nplanes >= 2 else 2, NP, W), jnp.float32),
        mesh=_mesh(),
        compiler_params=pltpu.CompilerParams(use_tc_tiling_on_sc=False),
        scratch_types=(
            [pltpu.VMEM((cpt, 128), jnp.int32),
             pltpu.VMEM((cpt, 128), jnp.int32)] +
            [pltpu.VMEM((128, W), jnp.float32) for _ in range(4)] +
            [pltpu.VMEM_SHARED((NP, W), jnp.float32)] +
            [pltpu.SemaphoreType.DMA for _ in range(8)]
        ),
    )(functools.partial(_agg_body, nplanes))


def _norm_body(deg_ref, x_ref, s_ref, d_ref, g0_ref):
    s = lax.rsqrt(jnp.maximum(deg_ref[0, :, 0:1], 1.0))
    d = lax.rsqrt(jnp.maximum(deg_ref[1, :, 0:1], 1.0))
    s_ref[...] = s
    d_ref[...] = d
    xs = x_ref[...] * s
    g0_ref[0] = xs[:, 0:W]
    g0_ref[1] = xs[:, W:2 * W]


def _norm_call(deg, x_pad):
    nb = NP // BN
    return pl.pallas_call(
        _norm_body,
        grid=(nb,),
        in_specs=[
            pl.BlockSpec((NC, BN, 16), lambda i: (0, i, 0)),
            pl.BlockSpec((BN, 128), lambda i: (i, 0)),
        ],
        out_specs=(
            pl.BlockSpec((BN, 1), lambda i: (i, 0)),
            pl.BlockSpec((BN, 1), lambda i: (i, 0)),
            pl.BlockSpec((2, BN, W), lambda i: (0, i, 0)),
        ),
        out_shape=(
            jax.ShapeDtypeStruct((NP, 1), jnp.float32),
            jax.ShapeDtypeStruct((NP, 1), jnp.float32),
            jax.ShapeDtypeStruct((2, NP, W), jnp.float32),
        ),
    )(deg, x_pad)


def _matmul_planes(a_ref, w_ref, np_in):
    acc = jnp.dot(a_ref[0], w_ref[0:W, :], preferred_element_type=jnp.float32)
    for p in range(1, np_in):
        acc = acc + jnp.dot(a_ref[p], w_ref[p * W:(p + 1) * W, :],
                            preferred_element_type=jnp.float32)
    return acc


def _layer_body(np_in, partial_sum, a_ref, w_ref, b_ref, s_ref, d_ref,
                out_ref):
    if partial_sum:
        a0 = a_ref[0] + a_ref[1]
        acc = jnp.dot(a0, w_ref[0:W, :], preferred_element_type=jnp.float32)
    else:
        acc = _matmul_planes(a_ref, w_ref, np_in)
    h = s_ref[...] * jnp.maximum(d_ref[...] * acc + b_ref[...], 0.0)
    out_ref[0] = h[:, 0:W]
    out_ref[1] = h[:, W:2 * W]


def _layer_call(np_in, partial_sum, a, w, b, sv, dv):
    nb = NP // BN
    kdim = w.shape[0]
    return pl.pallas_call(
        functools.partial(_layer_body, np_in, partial_sum),
        grid=(nb, 2),
        in_specs=[
            pl.BlockSpec((a.shape[0], BN, W), lambda i, q: (0, i, 0)),
            pl.BlockSpec((kdim, 128), lambda i, q: (0, q)),
            pl.BlockSpec((1, 128), lambda i, q: (0, q)),
            pl.BlockSpec((BN, 1), lambda i, q: (i, 0)),
            pl.BlockSpec((BN, 1), lambda i, q: (i, 0)),
        ],
        out_specs=pl.BlockSpec((2, BN, W), lambda i, q: (q, i, 0)),
        out_shape=jax.ShapeDtypeStruct((4, NP, W), jnp.float32),
    )(a, w, b, sv, dv)


def _tail_body(a_ref, w3_ref, b3_ref, w4_ref, s_ref, d_ref, z_ref):
    sv = s_ref[...]
    dv = d_ref[...]
    w3 = w3_ref[...]
    z = None
    for q in (0, 1):
        acc = None
        for p in range(4):
            t = jnp.dot(a_ref[p], w3[p * W:(p + 1) * W, q * 128:(q + 1) * 128],
                        preferred_element_type=jnp.float32)
            acc = t if acc is None else acc + t
        h = sv * jnp.maximum(dv * acc + b3_ref[0:1, q * 128:(q + 1) * 128],
                             0.0)
        zq = jnp.dot(h, w4_ref[q * 128:(q + 1) * 128, :],
                     preferred_element_type=jnp.float32)
        z = zq if z is None else z + zq
    z_ref[...] = z


def _tail_call(a, w3, b3, w4p, sv, dv):
    nb = NP // BN
    return pl.pallas_call(
        _tail_body,
        grid=(nb,),
        in_specs=[
            pl.BlockSpec((4, BN, W), lambda i: (0, i, 0)),
            pl.BlockSpec((256, 256), lambda i: (0, 0)),
            pl.BlockSpec((1, 256), lambda i: (0, 0)),
            pl.BlockSpec((256, 64), lambda i: (0, 0)),
            pl.BlockSpec((BN, 1), lambda i: (i, 0)),
            pl.BlockSpec((BN, 1), lambda i: (i, 0)),
        ],
        out_specs=pl.BlockSpec((BN, 64), lambda i: (i, 0)),
        out_shape=jax.ShapeDtypeStruct((NP, 64), jnp.float32),
    )(a, w3, b3, w4p, sv, dv)


def _final_body(z_ref, d_ref, b4_ref, out_ref):
    out_ref[...] = d_ref[...] * (z_ref[0] + z_ref[1]) + b4_ref[...]


def _final_call(z, dv, b4p):
    nb = NP // BN
    return pl.pallas_call(
        _final_body,
        grid=(nb,),
        in_specs=[
            pl.BlockSpec((NC, BN, 64), lambda i: (0, i, 0)),
            pl.BlockSpec((BN, 1), lambda i: (i, 0)),
            pl.BlockSpec((1, 64), lambda i: (0, 0)),
        ],
        out_specs=pl.BlockSpec((BN, 64), lambda i: (i, 0)),
        out_shape=jax.ShapeDtypeStruct((NP, 64), jnp.float32),
    )(z, dv, b4p)


def kernel(x, edge_index, W0, b0, W1, b1, W2, b2, W3, b3, W4, b4):
    src = edge_index[0].astype(jnp.int32)
    dst = edge_index[1].astype(jnp.int32)
    pad = EP - E
    sweep = jnp.arange(pad, dtype=jnp.int32)
    pad_gather = sweep * 67 % N
    pad_dump = N + sweep % (NP - N)
    src_g = jnp.concatenate([src, pad_gather]).reshape(ER, 128)
    src_d = jnp.concatenate([src, pad_dump]).reshape(ER, 128)
    dst_p = jnp.concatenate([dst, pad_dump]).reshape(ER, 128)

    x_pad = jnp.pad(x, ((0, NP - N), (0, 0)))
    b0r = b0.reshape(1, 256)
    b1r = b1.reshape(1, 256)
    b2r = b2.reshape(1, 256)
    b3r = b3.reshape(1, 256)
    w4p = jnp.pad(W4, ((0, 0), (0, 64 - W4.shape[1])))
    b4p = jnp.pad(b4, (0, 64 - b4.shape[0])).reshape(1, 64)

    ones16 = jnp.ones((128, 16), jnp.float32)
    zeros16 = jnp.zeros((RPT, 16), jnp.float32)
    zerosW = jnp.zeros((RPT, W), jnp.float32)

    deg = _make_deg()(src_d, dst_p, ones16, zeros16)
    sv, dv, g0 = _norm_call(deg, x_pad)

    a0 = _make_agg(2)(g0, src_g, dst_p, zerosW)
    g1 = _layer_call(2, False, a0, W0, b0r, sv, dv)
    a1 = _make_agg(4)(g1, src_g, dst_p, zerosW)
    g2 = _layer_call(4, False, a1, W1, b1r, sv, dv)
    a2 = _make_agg(4)(g2, src_g, dst_p, zerosW)
    g3 = _layer_call(4, False, a2, W2, b2r, sv, dv)
    a3 = _make_agg(4)(g3, src_g, dst_p, zerosW)
    z = _tail_call(a3, W3, b3r, w4p, sv, dv)
    za = _make_agg(1)(z, src_g, dst_p, zerosW)
    out = _final_call(za, dv, b4p)
    return out[:N, :40]


# speedup vs baseline: 9.4057x; 1.0387x over previous
import functools

import jax
import jax.numpy as jnp
from jax import lax
from jax.experimental import pallas as pl
from jax.experimental.pallas import tpu as pltpu
from jax.experimental.pallas import tpu_sc as plsc

N = 10000
NP = 10240
E = 320000
EP = 327680
ER = 2560
RPT = NP // 16
NC = 2
NS = 16
W = 64
BN = 512


@functools.cache
def _mesh():
    return plsc.VectorSubcoreMesh(
        core_axis_name="c", subcore_axis_name="s",
        num_cores=NC, num_subcores=NS)


def _deg_kernel(src_hbm, dst_hbm, ones_hbm, zeros_hbm, out_hbm,
                idx_v, ones_v, deg_sh, ds0, ds1, ds2, ds3):
    dsem = (ds0, ds1, ds2, ds3)
    c = lax.axis_index("c")
    s = lax.axis_index("s")
    cpt = ER // NS

    @pl.when(c == 0)
    def _():
        pltpu.sync_copy(src_hbm.at[pl.ds(s * cpt, cpt)], idx_v)

    @pl.when(c == 1)
    def _():
        pltpu.sync_copy(dst_hbm.at[pl.ds(s * cpt, cpt)], idx_v)

    pltpu.sync_copy(ones_hbm, ones_v)
    pltpu.sync_copy(zeros_hbm, deg_sh.at[pl.ds(s * RPT, RPT)])
    plsc.subcore_barrier()

    def body(g, carry):
        cps = [pltpu.async_copy(ones_v, deg_sh.at[idx_v.at[4 * g + b]],
                                dsem[b], add=True) for b in range(4)]
        for cp in cps:
            cp.wait()
        return carry

    lax.fori_loop(0, cpt // 4, body, 0)
    plsc.subcore_barrier()
    pltpu.sync_copy(deg_sh.at[pl.ds(s * RPT, RPT)],
                    out_hbm.at[c, pl.ds(s * RPT, RPT)])


def _make_deg():
    return functools.partial(
        pl.kernel,
        out_type=jax.ShapeDtypeStruct((NC, NP, 16), jnp.float32),
        mesh=_mesh(),
        compiler_params=pltpu.CompilerParams(use_tc_tiling_on_sc=False),
        scratch_types=[
            pltpu.VMEM((ER // NS, 128), jnp.int32),
            pltpu.VMEM((128, 16), jnp.float32),
            pltpu.VMEM_SHARED((NP, 16), jnp.float32),
            pltpu.SemaphoreType.DMA,
            pltpu.SemaphoreType.DMA,
            pltpu.SemaphoreType.DMA,
            pltpu.SemaphoreType.DMA,
        ],
    )(_deg_kernel)


def _agg_body(nplanes, g_hbm, src_hbm, dst_hbm, zeros_hbm, out_hbm,
              src_v, dst_v, r0, r1, r2, r3, agg_sh,
              gs0, gs1, gs2, gs3, ss0, ss1, ss2, ss3):
    rows = (r0, r1, r2, r3)
    gsem = (gs0, gs1, gs2, gs3)
    ssem = (ss0, ss1, ss2, ss3)
    c = lax.axis_index("c")
    s = lax.axis_index("s")
    if nplanes >= 2:
        cpt = ER // NS
        base = s * cpt
    else:
        cpt = ER // (NC * NS)
        base = (c * NS + s) * cpt
    pltpu.sync_copy(src_hbm.at[pl.ds(base, cpt)], src_v)
    pltpu.sync_copy(dst_hbm.at[pl.ds(base, cpt)], dst_v)

    def one_plane(table, out_slot):
        pltpu.sync_copy(zeros_hbm, agg_sh.at[pl.ds(s * RPT, RPT)])
        plsc.subcore_barrier()

        def gather(ch, b):
            return pltpu.async_copy(table.at[src_v.at[ch]], rows[b], gsem[b])

        def gather_wait(ch, b):
            pltpu.make_async_copy(table.at[src_v.at[ch]], rows[b],
                                  gsem[b]).wait()

        def scatter(ch, b):
            return pltpu.async_copy(rows[b], agg_sh.at[dst_v.at[ch]],
                                    ssem[b], add=True)

        for b in range(4):
            gather(b, b)

        def body(g, carry):
            cps = []
            for b in range(4):
                gather_wait(4 * g + b, b)
                cps.append(scatter(4 * g + b, b))
            for b in range(4):
                cps[b].wait()
                gather(4 * g + 4 + b, b)
            return carry

        lax.fori_loop(0, cpt // 4 - 1, body, 0)
        cps = []
        for b in range(4):
            ch = cpt - 4 + b
            gather_wait(ch, b)
            cps.append(scatter(ch, b))
        for cp in cps:
            cp.wait()
        plsc.subcore_barrier()
        pltpu.sync_copy(agg_sh.at[pl.ds(s * RPT, RPT)],
                        out_hbm.at[out_slot, pl.ds(s * RPT, RPT)])
        plsc.subcore_barrier()

    if nplanes == 1:
        one_plane(g_hbm, c)
    else:
        for pp in range(nplanes // 2):
            plane = c * (nplanes // 2) + pp
            one_plane(g_hbm.at[plane], plane)


def _make_agg(nplanes):
    cpt = ER // NS if nplanes >= 2 else ER // (NC * NS)
    return functools.partial(
        pl.kernel,
        out_type=jax.ShapeDtypeStruct(
            (nplanes if nplanes >= 2 else 2, NP, W), jnp.float32),
        mesh=_mesh(),
        compiler_params=pltpu.CompilerParams(use_tc_tiling_on_sc=False),
        scratch_types=(
            [pltpu.VMEM((cpt, 128), jnp.int32),
             pltpu.VMEM((cpt, 128), jnp.int32)] +
            [pltpu.VMEM((128, W), jnp.float32) for _ in range(4)] +
            [pltpu.VMEM_SHARED((NP, W), jnp.float32)] +
            [pltpu.SemaphoreType.DMA for _ in range(8)]
        ),
    )(functools.partial(_agg_body, nplanes))


def _norm_body(deg_ref, x_ref, s_ref, d_ref, g0_ref):
    s = lax.rsqrt(jnp.maximum(deg_ref[0, :, 0:1], 1.0))
    d = lax.rsqrt(jnp.maximum(deg_ref[1, :, 0:1], 1.0))
    s_ref[...] = s
    d_ref[...] = d
    xs = x_ref[...] * s
    g0_ref[0] = xs[:, 0:W]
    g0_ref[1] = xs[:, W:2 * W]


def _norm_call(deg, x_pad):
    nb = NP // BN
    return pl.pallas_call(
        _norm_body,
        grid=(nb,),
        in_specs=[
            pl.BlockSpec((NC, BN, 16), lambda i: (0, i, 0)),
            pl.BlockSpec((BN, 128), lambda i: (i, 0)),
        ],
        out_specs=(
            pl.BlockSpec((BN, 1), lambda i: (i, 0)),
            pl.BlockSpec((BN, 1), lambda i: (i, 0)),
            pl.BlockSpec((2, BN, W), lambda i: (0, i, 0)),
        ),
        out_shape=(
            jax.ShapeDtypeStruct((NP, 1), jnp.float32),
            jax.ShapeDtypeStruct((NP, 1), jnp.float32),
            jax.ShapeDtypeStruct((2, NP, W), jnp.float32),
        ),
    )(deg, x_pad)


def _matmul_planes(a_ref, w_ref, np_in):
    acc = jnp.dot(a_ref[0], w_ref[0:W, :], preferred_element_type=jnp.float32)
    for p in range(1, np_in):
        acc = acc + jnp.dot(a_ref[p], w_ref[p * W:(p + 1) * W, :],
                            preferred_element_type=jnp.float32)
    return acc


def _layer_body(np_in, a_ref, w_ref, b_ref, s_ref, d_ref, out_ref):
    sv = s_ref[...]
    dv = d_ref[...]
    w = w_ref[...]
    for q in (0, 1):
        acc = None
        for p in range(np_in):
            t = jnp.dot(a_ref[p], w[p * W:(p + 1) * W, q * 128:(q + 1) * 128],
                        preferred_element_type=jnp.float32)
            acc = t if acc is None else acc + t
        h = sv * jnp.maximum(dv * acc + b_ref[0:1, q * 128:(q + 1) * 128],
                             0.0)
        out_ref[2 * q] = h[:, 0:W]
        out_ref[2 * q + 1] = h[:, W:2 * W]


def _layer_call(np_in, a, w, b, sv, dv):
    nb = NP // BN
    kdim = w.shape[0]
    return pl.pallas_call(
        functools.partial(_layer_body, np_in),
        grid=(nb,),
        in_specs=[
            pl.BlockSpec((a.shape[0], BN, W), lambda i: (0, i, 0)),
            pl.BlockSpec((kdim, 256), lambda i: (0, 0)),
            pl.BlockSpec((1, 256), lambda i: (0, 0)),
            pl.BlockSpec((BN, 1), lambda i: (i, 0)),
            pl.BlockSpec((BN, 1), lambda i: (i, 0)),
        ],
        out_specs=pl.BlockSpec((4, BN, W), lambda i: (0, i, 0)),
        out_shape=jax.ShapeDtypeStruct((4, NP, W), jnp.float32),
    )(a, w, b, sv, dv)


def _tail_body(a_ref, w3_ref, b3_ref, w4_ref, s_ref, d_ref, z_ref):
    sv = s_ref[...]
    dv = d_ref[...]
    w3 = w3_ref[...]
    z = None
    for q in (0, 1):
        acc = None
        for p in range(4):
            t = jnp.dot(a_ref[p], w3[p * W:(p + 1) * W, q * 128:(q + 1) * 128],
                        preferred_element_type=jnp.float32)
            acc = t if acc is None else acc + t
        h = sv * jnp.maximum(dv * acc + b3_ref[0:1, q * 128:(q + 1) * 128],
                             0.0)
        zq = jnp.dot(h, w4_ref[q * 128:(q + 1) * 128, :],
                     preferred_element_type=jnp.float32)
        z = zq if z is None else z + zq
    z_ref[...] = z


def _tail_call(a, w3, b3, w4p, sv, dv):
    nb = NP // BN
    return pl.pallas_call(
        _tail_body,
        grid=(nb,),
        in_specs=[
            pl.BlockSpec((4, BN, W), lambda i: (0, i, 0)),
            pl.BlockSpec((256, 256), lambda i: (0, 0)),
            pl.BlockSpec((1, 256), lambda i: (0, 0)),
            pl.BlockSpec((256, 64), lambda i: (0, 0)),
            pl.BlockSpec((BN, 1), lambda i: (i, 0)),
            pl.BlockSpec((BN, 1), lambda i: (i, 0)),
        ],
        out_specs=pl.BlockSpec((BN, 64), lambda i: (i, 0)),
        out_shape=jax.ShapeDtypeStruct((NP, 64), jnp.float32),
    )(a, w3, b3, w4p, sv, dv)


def _final_body(z_ref, d_ref, b4_ref, out_ref):
    out_ref[...] = d_ref[...] * (z_ref[0] + z_ref[1]) + b4_ref[...]


def _final_call(z, dv, b4p):
    nb = NP // BN
    return pl.pallas_call(
        _final_body,
        grid=(nb,),
        in_specs=[
            pl.BlockSpec((NC, BN, 64), lambda i: (0, i, 0)),
            pl.BlockSpec((BN, 1), lambda i: (i, 0)),
            pl.BlockSpec((1, 64), lambda i: (0, 0)),
        ],
        out_specs=pl.BlockSpec((BN, 64), lambda i: (i, 0)),
        out_shape=jax.ShapeDtypeStruct((NP, 64), jnp.float32),
    )(z, dv, b4p)


def kernel(x, edge_index, W0, b0, W1, b1, W2, b2, W3, b3, W4, b4):
    src = edge_index[0].astype(jnp.int32)
    dst = edge_index[1].astype(jnp.int32)
    pad = EP - E
    sweep = jnp.arange(pad, dtype=jnp.int32)
    pad_gather = sweep * 67 % N
    pad_dump = N + sweep % (NP - N)
    src_g = jnp.concatenate([src, pad_gather]).reshape(ER, 128)
    src_d = jnp.concatenate([src, pad_dump]).reshape(ER, 128)
    dst_p = jnp.concatenate([dst, pad_dump]).reshape(ER, 128)

    x_pad = jnp.pad(x, ((0, NP - N), (0, 0)))
    b0r = b0.reshape(1, 256)
    b1r = b1.reshape(1, 256)
    b2r = b2.reshape(1, 256)
    b3r = b3.reshape(1, 256)
    w4p = jnp.pad(W4, ((0, 0), (0, 64 - W4.shape[1])))
    b4p = jnp.pad(b4, (0, 64 - b4.shape[0])).reshape(1, 64)

    ones16 = jnp.ones((128, 16), jnp.float32)
    zeros16 = jnp.zeros((RPT, 16), jnp.float32)
    zerosW = jnp.zeros((RPT, W), jnp.float32)

    deg = _make_deg()(src_d, dst_p, ones16, zeros16)
    sv, dv, g0 = _norm_call(deg, x_pad)

    a0 = _make_agg(2)(g0, src_g, dst_p, zerosW)
    g1 = _layer_call(2, a0, W0, b0r, sv, dv)
    a1 = _make_agg(4)(g1, src_g, dst_p, zerosW)
    g2 = _layer_call(4, a1, W1, b1r, sv, dv)
    a2 = _make_agg(4)(g2, src_g, dst_p, zerosW)
    g3 = _layer_call(4, a2, W2, b2r, sv, dv)
    a3 = _make_agg(4)(g3, src_g, dst_p, zerosW)
    z = _tail_call(a3, W3, b3r, w4p, sv, dv)
    za = _make_agg(1)(z, src_g, dst_p, zerosW)
    out = _final_call(za, dv, b4p)
    return out[:N, :40]


# speedup vs baseline: 9.5750x; 1.0180x over previous
import functools

import jax
import jax.numpy as jnp
from jax import lax
from jax.experimental import pallas as pl
from jax.experimental.pallas import tpu as pltpu
from jax.experimental.pallas import tpu_sc as plsc

N = 10000
NP = 10240
E = 320000
EP = 327680
ER = 2560
RPT = NP // 16
NC = 2
NS = 16
W = 64
BN = 512


@functools.cache
def _mesh():
    return plsc.VectorSubcoreMesh(
        core_axis_name="c", subcore_axis_name="s",
        num_cores=NC, num_subcores=NS)


def _deg_kernel(src_hbm, dst_hbm, ones_hbm, zeros_hbm, out_hbm,
                idx_v, ones_v, deg_sh, ds0, ds1, ds2, ds3):
    dsem = (ds0, ds1, ds2, ds3)
    c = lax.axis_index("c")
    s = lax.axis_index("s")
    cpt = ER // NS

    @pl.when(c == 0)
    def _():
        pltpu.sync_copy(src_hbm.at[pl.ds(s * cpt, cpt)], idx_v)

    @pl.when(c == 1)
    def _():
        pltpu.sync_copy(dst_hbm.at[pl.ds(s * cpt, cpt)], idx_v)

    pltpu.sync_copy(ones_hbm, ones_v)
    pltpu.sync_copy(zeros_hbm, deg_sh.at[pl.ds(s * RPT, RPT)])
    plsc.subcore_barrier()

    def body(g, carry):
        cps = [pltpu.async_copy(ones_v, deg_sh.at[idx_v.at[4 * g + b]],
                                dsem[b], add=True) for b in range(4)]
        for cp in cps:
            cp.wait()
        return carry

    lax.fori_loop(0, cpt // 4, body, 0)
    plsc.subcore_barrier()
    pltpu.sync_copy(deg_sh.at[pl.ds(s * RPT, RPT)],
                    out_hbm.at[c, pl.ds(s * RPT, RPT)])


def _make_deg():
    return functools.partial(
        pl.kernel,
        out_type=jax.ShapeDtypeStruct((NC, NP, 16), jnp.float32),
        mesh=_mesh(),
        compiler_params=pltpu.CompilerParams(use_tc_tiling_on_sc=False),
        scratch_types=[
            pltpu.VMEM((ER // NS, 128), jnp.int32),
            pltpu.VMEM((128, 16), jnp.float32),
            pltpu.VMEM_SHARED((NP, 16), jnp.float32),
            pltpu.SemaphoreType.DMA,
            pltpu.SemaphoreType.DMA,
            pltpu.SemaphoreType.DMA,
            pltpu.SemaphoreType.DMA,
        ],
    )(_deg_kernel)


NB = 5


def _agg_body(nplanes, g_hbm, src_hbm, dst_hbm, zeros_hbm, out_hbm,
              src_v, dst_v, *bufs):
    rows = bufs[:NB]
    agg_sh = bufs[NB]
    gsem = bufs[NB + 1:2 * NB + 1]
    ssem = bufs[2 * NB + 1:3 * NB + 1]
    c = lax.axis_index("c")
    s = lax.axis_index("s")
    if nplanes >= 2:
        cpt = ER // NS
        base = s * cpt
    else:
        cpt = ER // (NC * NS)
        base = (c * NS + s) * cpt
    pltpu.sync_copy(src_hbm.at[pl.ds(base, cpt)], src_v)
    pltpu.sync_copy(dst_hbm.at[pl.ds(base, cpt)], dst_v)

    def one_plane(table, out_slot):
        pltpu.sync_copy(zeros_hbm, agg_sh.at[pl.ds(s * RPT, RPT)])
        plsc.subcore_barrier()

        def gather(ch, b):
            return pltpu.async_copy(table.at[src_v.at[ch]], rows[b], gsem[b])

        def gather_wait(ch, b):
            pltpu.make_async_copy(table.at[src_v.at[ch]], rows[b],
                                  gsem[b]).wait()

        def scatter(ch, b):
            return pltpu.async_copy(rows[b], agg_sh.at[dst_v.at[ch]],
                                    ssem[b], add=True)

        for b in range(NB):
            gather(b, b)

        def body(g, carry):
            cps = []
            for b in range(NB):
                gather_wait(NB * g + b, b)
                cps.append(scatter(NB * g + b, b))
            for b in range(NB):
                cps[b].wait()
                gather(NB * g + NB + b, b)
            return carry

        lax.fori_loop(0, cpt // NB - 1, body, 0)
        cps = []
        for b in range(NB):
            ch = cpt - NB + b
            gather_wait(ch, b)
            cps.append(scatter(ch, b))
        for cp in cps:
            cp.wait()
        plsc.subcore_barrier()
        pltpu.sync_copy(agg_sh.at[pl.ds(s * RPT, RPT)],
                        out_hbm.at[out_slot, pl.ds(s * RPT, RPT)])
        plsc.subcore_barrier()

    if nplanes == 1:
        one_plane(g_hbm, c)
    else:
        for pp in range(nplanes // 2):
            plane = c * (nplanes // 2) + pp
            one_plane(g_hbm.at[plane], plane)


def _make_agg(nplanes):
    cpt = ER // NS if nplanes >= 2 else ER // (NC * NS)
    return functools.partial(
        pl.kernel,
        out_type=jax.ShapeDtypeStruct(
            (nplanes if nplanes >= 2 else 2, NP, W), jnp.float32),
        mesh=_mesh(),
        compiler_params=pltpu.CompilerParams(use_tc_tiling_on_sc=False),
        scratch_types=(
            [pltpu.VMEM((cpt, 128), jnp.int32),
             pltpu.VMEM((cpt, 128), jnp.int32)] +
            [pltpu.VMEM((128, W), jnp.float32) for _ in range(NB)] +
            [pltpu.VMEM_SHARED((NP, W), jnp.float32)] +
            [pltpu.SemaphoreType.DMA for _ in range(2 * NB)]
        ),
    )(functools.partial(_agg_body, nplanes))


def _norm_body(deg_ref, x_ref, s_ref, d_ref, g0_ref):
    s = lax.rsqrt(jnp.maximum(deg_ref[0, :, 0:1], 1.0))
    d = lax.rsqrt(jnp.maximum(deg_ref[1, :, 0:1], 1.0))
    s_ref[...] = s
    d_ref[...] = d
    xs = x_ref[...] * s
    g0_ref[0] = xs[:, 0:W]
    g0_ref[1] = xs[:, W:2 * W]


def _norm_call(deg, x_pad):
    nb = NP // BN
    return pl.pallas_call(
        _norm_body,
        grid=(nb,),
        in_specs=[
            pl.BlockSpec((NC, BN, 16), lambda i: (0, i, 0)),
            pl.BlockSpec((BN, 128), lambda i: (i, 0)),
        ],
        out_specs=(
            pl.BlockSpec((BN, 1), lambda i: (i, 0)),
            pl.BlockSpec((BN, 1), lambda i: (i, 0)),
            pl.BlockSpec((2, BN, W), lambda i: (0, i, 0)),
        ),
        out_shape=(
            jax.ShapeDtypeStruct((NP, 1), jnp.float32),
            jax.ShapeDtypeStruct((NP, 1), jnp.float32),
            jax.ShapeDtypeStruct((2, NP, W), jnp.float32),
        ),
    )(deg, x_pad)


def _matmul_planes(a_ref, w_ref, np_in):
    acc = jnp.dot(a_ref[0], w_ref[0:W, :], preferred_element_type=jnp.float32)
    for p in range(1, np_in):
        acc = acc + jnp.dot(a_ref[p], w_ref[p * W:(p + 1) * W, :],
                            preferred_element_type=jnp.float32)
    return acc


def _layer_body(np_in, a_ref, w_ref, b_ref, s_ref, d_ref, out_ref):
    sv = s_ref[...]
    dv = d_ref[...]
    w = w_ref[...]
    for q in (0, 1):
        acc = None
        for p in range(np_in):
            t = jnp.dot(a_ref[p], w[p * W:(p + 1) * W, q * 128:(q + 1) * 128],
                        preferred_element_type=jnp.float32)
            acc = t if acc is None else acc + t
        h = sv * jnp.maximum(dv * acc + b_ref[0:1, q * 128:(q + 1) * 128],
                             0.0)
        out_ref[2 * q] = h[:, 0:W]
        out_ref[2 * q + 1] = h[:, W:2 * W]


def _layer_call(np_in, a, w, b, sv, dv):
    nb = NP // BN
    kdim = w.shape[0]
    return pl.pallas_call(
        functools.partial(_layer_body, np_in),
        grid=(nb,),
        in_specs=[
            pl.BlockSpec((a.shape[0], BN, W), lambda i: (0, i, 0)),
            pl.BlockSpec((kdim, 256), lambda i: (0, 0)),
            pl.BlockSpec((1, 256), lambda i: (0, 0)),
            pl.BlockSpec((BN, 1), lambda i: (i, 0)),
            pl.BlockSpec((BN, 1), lambda i: (i, 0)),
        ],
        out_specs=pl.BlockSpec((4, BN, W), lambda i: (0, i, 0)),
        out_shape=jax.ShapeDtypeStruct((4, NP, W), jnp.float32),
    )(a, w, b, sv, dv)


def _tail_body(a_ref, w3_ref, b3_ref, w4_ref, s_ref, d_ref, z_ref):
    sv = s_ref[...]
    dv = d_ref[...]
    w3 = w3_ref[...]
    z = None
    for q in (0, 1):
        acc = None
        for p in range(4):
            t = jnp.dot(a_ref[p], w3[p * W:(p + 1) * W, q * 128:(q + 1) * 128],
                        preferred_element_type=jnp.float32)
            acc = t if acc is None else acc + t
        h = sv * jnp.maximum(dv * acc + b3_ref[0:1, q * 128:(q + 1) * 128],
                             0.0)
        zq = jnp.dot(h, w4_ref[q * 128:(q + 1) * 128, :],
                     preferred_element_type=jnp.float32)
        z = zq if z is None else z + zq
    z_ref[...] = z


def _tail_call(a, w3, b3, w4p, sv, dv):
    nb = NP // BN
    return pl.pallas_call(
        _tail_body,
        grid=(nb,),
        in_specs=[
            pl.BlockSpec((4, BN, W), lambda i: (0, i, 0)),
            pl.BlockSpec((256, 256), lambda i: (0, 0)),
            pl.BlockSpec((1, 256), lambda i: (0, 0)),
            pl.BlockSpec((256, 64), lambda i: (0, 0)),
            pl.BlockSpec((BN, 1), lambda i: (i, 0)),
            pl.BlockSpec((BN, 1), lambda i: (i, 0)),
        ],
        out_specs=pl.BlockSpec((BN, 64), lambda i: (i, 0)),
        out_shape=jax.ShapeDtypeStruct((NP, 64), jnp.float32),
    )(a, w3, b3, w4p, sv, dv)


def _final_body(z_ref, d_ref, b4_ref, out_ref):
    out_ref[...] = d_ref[...] * (z_ref[0] + z_ref[1]) + b4_ref[...]


def _final_call(z, dv, b4p):
    nb = NP // BN
    return pl.pallas_call(
        _final_body,
        grid=(nb,),
        in_specs=[
            pl.BlockSpec((NC, BN, 64), lambda i: (0, i, 0)),
            pl.BlockSpec((BN, 1), lambda i: (i, 0)),
            pl.BlockSpec((1, 64), lambda i: (0, 0)),
        ],
        out_specs=pl.BlockSpec((BN, 64), lambda i: (i, 0)),
        out_shape=jax.ShapeDtypeStruct((NP, 64), jnp.float32),
    )(z, dv, b4p)


def kernel(x, edge_index, W0, b0, W1, b1, W2, b2, W3, b3, W4, b4):
    src = edge_index[0].astype(jnp.int32)
    dst = edge_index[1].astype(jnp.int32)
    pad = EP - E
    sweep = jnp.arange(pad, dtype=jnp.int32)
    pad_gather = sweep * 67 % N
    pad_dump = N + sweep % (NP - N)
    src_g = jnp.concatenate([src, pad_gather]).reshape(ER, 128)
    src_d = jnp.concatenate([src, pad_dump]).reshape(ER, 128)
    dst_p = jnp.concatenate([dst, pad_dump]).reshape(ER, 128)

    x_pad = jnp.pad(x, ((0, NP - N), (0, 0)))
    b0r = b0.reshape(1, 256)
    b1r = b1.reshape(1, 256)
    b2r = b2.reshape(1, 256)
    b3r = b3.reshape(1, 256)
    w4p = jnp.pad(W4, ((0, 0), (0, 64 - W4.shape[1])))
    b4p = jnp.pad(b4, (0, 64 - b4.shape[0])).reshape(1, 64)

    ones16 = jnp.ones((128, 16), jnp.float32)
    zeros16 = jnp.zeros((RPT, 16), jnp.float32)
    zerosW = jnp.zeros((RPT, W), jnp.float32)

    deg = _make_deg()(src_d, dst_p, ones16, zeros16)
    sv, dv, g0 = _norm_call(deg, x_pad)

    a0 = _make_agg(2)(g0, src_g, dst_p, zerosW)
    g1 = _layer_call(2, a0, W0, b0r, sv, dv)
    a1 = _make_agg(4)(g1, src_g, dst_p, zerosW)
    g2 = _layer_call(4, a1, W1, b1r, sv, dv)
    a2 = _make_agg(4)(g2, src_g, dst_p, zerosW)
    g3 = _layer_call(4, a2, W2, b2r, sv, dv)
    a3 = _make_agg(4)(g3, src_g, dst_p, zerosW)
    z = _tail_call(a3, W3, b3r, w4p, sv, dv)
    za = _make_agg(1)(z, src_g, dst_p, zerosW)
    out = _final_call(za, dv, b4p)
    return out[:N, :40]


# speedup vs baseline: 10.2747x; 1.0731x over previous
import functools

import jax
import jax.numpy as jnp
from jax import lax
from jax.experimental import pallas as pl
from jax.experimental.pallas import tpu as pltpu
from jax.experimental.pallas import tpu_sc as plsc

N = 10000
NP = 10240
E = 320000
EP = 327680
ER = 2560
RPT = NP // 16
NC = 2
NS = 16
W = 64
BN = 512


@functools.cache
def _mesh():
    return plsc.VectorSubcoreMesh(
        core_axis_name="c", subcore_axis_name="s",
        num_cores=NC, num_subcores=NS)


def _deg_kernel(src_hbm, dst_hbm, ones_hbm, zeros_hbm, out_hbm,
                idx_v, ones_v, deg_sh, ds0, ds1, ds2, ds3):
    dsem = (ds0, ds1, ds2, ds3)
    c = lax.axis_index("c")
    s = lax.axis_index("s")
    cpt = ER // NS

    @pl.when(c == 0)
    def _():
        pltpu.sync_copy(src_hbm.at[pl.ds(s * cpt, cpt)], idx_v)

    @pl.when(c == 1)
    def _():
        pltpu.sync_copy(dst_hbm.at[pl.ds(s * cpt, cpt)], idx_v)

    pltpu.sync_copy(ones_hbm, ones_v)
    pltpu.sync_copy(zeros_hbm, deg_sh.at[pl.ds(s * RPT, RPT)])
    plsc.subcore_barrier()

    def body(g, carry):
        cps = [pltpu.async_copy(ones_v, deg_sh.at[idx_v.at[4 * g + b]],
                                dsem[b], add=True) for b in range(4)]
        for cp in cps:
            cp.wait()
        return carry

    lax.fori_loop(0, cpt // 4, body, 0)
    plsc.subcore_barrier()
    pltpu.sync_copy(deg_sh.at[pl.ds(s * RPT, RPT)],
                    out_hbm.at[c, pl.ds(s * RPT, RPT)])


def _make_deg():
    return functools.partial(
        pl.kernel,
        out_type=jax.ShapeDtypeStruct((NC, NP, 16), jnp.float32),
        mesh=_mesh(),
        compiler_params=pltpu.CompilerParams(use_tc_tiling_on_sc=False),
        scratch_types=[
            pltpu.VMEM((ER // NS, 128), jnp.int32),
            pltpu.VMEM((128, 16), jnp.float32),
            pltpu.VMEM_SHARED((NP, 16), jnp.float32),
            pltpu.SemaphoreType.DMA,
            pltpu.SemaphoreType.DMA,
            pltpu.SemaphoreType.DMA,
            pltpu.SemaphoreType.DMA,
        ],
    )(_deg_kernel)


NB = 5


def _agg_body(nplanes, g_hbm, src_hbm, dst_hbm, zeros_hbm, out_hbm,
              src_v, dst_v, *bufs):
    rows = bufs[:NB]
    agg_sh = bufs[NB]
    gsem = bufs[NB + 1:2 * NB + 1]
    ssem = bufs[2 * NB + 1:3 * NB + 1]
    c = lax.axis_index("c")
    s = lax.axis_index("s")
    if nplanes >= 2:
        cpt = ER // NS
        base = s * cpt
    else:
        cpt = ER // (NC * NS)
        base = (c * NS + s) * cpt
    pltpu.sync_copy(src_hbm.at[pl.ds(base, cpt)], src_v)
    pltpu.sync_copy(dst_hbm.at[pl.ds(base, cpt)], dst_v)

    def one_plane(table, out_ref):
        pltpu.sync_copy(zeros_hbm, agg_sh.at[pl.ds(s * RPT, RPT)])
        plsc.subcore_barrier()

        def gather(ch, b):
            return pltpu.async_copy(table.at[src_v.at[ch]], rows[b], gsem[b])

        def gather_wait(ch, b):
            pltpu.make_async_copy(table.at[src_v.at[ch]], rows[b],
                                  gsem[b]).wait()

        def scatter(ch, b):
            return pltpu.async_copy(rows[b], agg_sh.at[dst_v.at[ch]],
                                    ssem[b], add=True)

        for b in range(NB):
            gather(b, b)

        def body(g, carry):
            cps = []
            for b in range(NB):
                gather_wait(NB * g + b, b)
                cps.append(scatter(NB * g + b, b))
            for b in range(NB):
                cps[b].wait()
                gather(NB * g + NB + b, b)
            return carry

        lax.fori_loop(0, cpt // NB - 1, body, 0)
        cps = []
        for b in range(NB):
            ch = cpt - NB + b
            gather_wait(ch, b)
            cps.append(scatter(ch, b))
        for cp in cps:
            cp.wait()
        plsc.subcore_barrier()
        pltpu.sync_copy(agg_sh.at[pl.ds(s * RPT, RPT)], out_ref)
        plsc.subcore_barrier()

    if nplanes == 1:
        one_plane(g_hbm,
                  out_hbm.at[pl.ds(s * RPT, RPT), pl.ds(64 * c, W)])
    elif nplanes == 2:
        one_plane(g_hbm.at[c],
                  out_hbm.at[pl.ds(s * RPT, RPT), pl.ds(64 * c, W)])
    else:
        for pp in range(nplanes // 2):
            one_plane(g_hbm.at[c * (nplanes // 2) + pp],
                      out_hbm.at[c, pl.ds(s * RPT, RPT), pl.ds(64 * pp, W)])


def _make_agg(nplanes):
    cpt = ER // NS if nplanes >= 2 else ER // (NC * NS)
    out_shape = ((NC, NP, 128) if nplanes == 4 else (NP, 128))
    return functools.partial(
        pl.kernel,
        out_type=jax.ShapeDtypeStruct(out_shape, jnp.float32),
        mesh=_mesh(),
        compiler_params=pltpu.CompilerParams(use_tc_tiling_on_sc=False),
        scratch_types=(
            [pltpu.VMEM((cpt, 128), jnp.int32),
             pltpu.VMEM((cpt, 128), jnp.int32)] +
            [pltpu.VMEM((128, W), jnp.float32) for _ in range(NB)] +
            [pltpu.VMEM_SHARED((NP, W), jnp.float32)] +
            [pltpu.SemaphoreType.DMA for _ in range(2 * NB)]
        ),
    )(functools.partial(_agg_body, nplanes))


def _norm_body(deg_ref, x_ref, s_ref, d_ref, g0_ref):
    s = lax.rsqrt(jnp.maximum(deg_ref[0, :, 0:1], 1.0))
    d = lax.rsqrt(jnp.maximum(deg_ref[1, :, 0:1], 1.0))
    s_ref[...] = s
    d_ref[...] = d
    xs = x_ref[...] * s
    g0_ref[0] = xs[:, 0:W]
    g0_ref[1] = xs[:, W:2 * W]


def _norm_call(deg, x_pad):
    nb = NP // BN
    return pl.pallas_call(
        _norm_body,
        grid=(nb,),
        in_specs=[
            pl.BlockSpec((NC, BN, 16), lambda i: (0, i, 0)),
            pl.BlockSpec((BN, 128), lambda i: (i, 0)),
        ],
        out_specs=(
            pl.BlockSpec((BN, 1), lambda i: (i, 0)),
            pl.BlockSpec((BN, 1), lambda i: (i, 0)),
            pl.BlockSpec((2, BN, W), lambda i: (0, i, 0)),
        ),
        out_shape=(
            jax.ShapeDtypeStruct((NP, 1), jnp.float32),
            jax.ShapeDtypeStruct((NP, 1), jnp.float32),
            jax.ShapeDtypeStruct((2, NP, W), jnp.float32),
        ),
    )(deg, x_pad)


def _layer_body(kin, a_ref, w_ref, b_ref, s_ref, d_ref, out_ref):
    sv = s_ref[...]
    dv = d_ref[...]
    w = w_ref[...]
    for q in (0, 1):
        acc = None
        for k in range(kin):
            a = a_ref[...] if kin == 1 else a_ref[k]
            t = jnp.dot(a, w[k * 128:(k + 1) * 128, q * 128:(q + 1) * 128],
                        preferred_element_type=jnp.float32)
            acc = t if acc is None else acc + t
        h = sv * jnp.maximum(dv * acc + b_ref[0:1, q * 128:(q + 1) * 128],
                             0.0)
        out_ref[2 * q] = h[:, 0:W]
        out_ref[2 * q + 1] = h[:, W:2 * W]


def _layer_call(a, w, b, sv, dv):
    nb = NP // BN
    kin = 1 if a.ndim == 2 else a.shape[0]
    a_spec = (pl.BlockSpec((BN, 128), lambda i: (i, 0)) if kin == 1
              else pl.BlockSpec((kin, BN, 128), lambda i: (0, i, 0)))
    kdim = w.shape[0]
    return pl.pallas_call(
        functools.partial(_layer_body, kin),
        grid=(nb,),
        in_specs=[
            a_spec,
            pl.BlockSpec((kdim, 256), lambda i: (0, 0)),
            pl.BlockSpec((1, 256), lambda i: (0, 0)),
            pl.BlockSpec((BN, 1), lambda i: (i, 0)),
            pl.BlockSpec((BN, 1), lambda i: (i, 0)),
        ],
        out_specs=pl.BlockSpec((4, BN, W), lambda i: (0, i, 0)),
        out_shape=jax.ShapeDtypeStruct((4, NP, W), jnp.float32),
    )(a, w, b, sv, dv)


def _tail_body(a_ref, w3_ref, b3_ref, w4_ref, s_ref, d_ref, z_ref):
    sv = s_ref[...]
    dv = d_ref[...]
    w3 = w3_ref[...]
    z = None
    for q in (0, 1):
        acc = None
        for k in range(2):
            t = jnp.dot(a_ref[k],
                        w3[k * 128:(k + 1) * 128, q * 128:(q + 1) * 128],
                        preferred_element_type=jnp.float32)
            acc = t if acc is None else acc + t
        h = sv * jnp.maximum(dv * acc + b3_ref[0:1, q * 128:(q + 1) * 128],
                             0.0)
        zq = jnp.dot(h, w4_ref[q * 128:(q + 1) * 128, :],
                     preferred_element_type=jnp.float32)
        z = zq if z is None else z + zq
    z_ref[...] = z


def _tail_call(a, w3, b3, w4p, sv, dv):
    nb = NP // BN
    return pl.pallas_call(
        _tail_body,
        grid=(nb,),
        in_specs=[
            pl.BlockSpec((NC, BN, 128), lambda i: (0, i, 0)),
            pl.BlockSpec((256, 256), lambda i: (0, 0)),
            pl.BlockSpec((1, 256), lambda i: (0, 0)),
            pl.BlockSpec((256, 64), lambda i: (0, 0)),
            pl.BlockSpec((BN, 1), lambda i: (i, 0)),
            pl.BlockSpec((BN, 1), lambda i: (i, 0)),
        ],
        out_specs=pl.BlockSpec((BN, 64), lambda i: (i, 0)),
        out_shape=jax.ShapeDtypeStruct((NP, 64), jnp.float32),
    )(a, w3, b3, w4p, sv, dv)


def _final_body(z_ref, d_ref, b4_ref, out_ref):
    z = z_ref[...]
    out_ref[...] = d_ref[...] * (z[:, 0:W] + z[:, W:2 * W]) + b4_ref[...]


def _final_call(z, dv, b4p):
    nb = NP // BN
    return pl.pallas_call(
        _final_body,
        grid=(nb,),
        in_specs=[
            pl.BlockSpec((BN, 128), lambda i: (i, 0)),
            pl.BlockSpec((BN, 1), lambda i: (i, 0)),
            pl.BlockSpec((1, 64), lambda i: (0, 0)),
        ],
        out_specs=pl.BlockSpec((BN, 64), lambda i: (i, 0)),
        out_shape=jax.ShapeDtypeStruct((NP, 64), jnp.float32),
    )(z, dv, b4p)


def kernel(x, edge_index, W0, b0, W1, b1, W2, b2, W3, b3, W4, b4):
    src = edge_index[0].astype(jnp.int32)
    dst = edge_index[1].astype(jnp.int32)
    pad = EP - E
    sweep = jnp.arange(pad, dtype=jnp.int32)
    pad_gather = sweep * 67 % N
    pad_dump = N + sweep % (NP - N)
    src_g = jnp.concatenate([src, pad_gather]).reshape(ER, 128)
    src_d = jnp.concatenate([src, pad_dump]).reshape(ER, 128)
    dst_p = jnp.concatenate([dst, pad_dump]).reshape(ER, 128)

    x_pad = jnp.pad(x, ((0, NP - N), (0, 0)))
    b0r = b0.reshape(1, 256)
    b1r = b1.reshape(1, 256)
    b2r = b2.reshape(1, 256)
    b3r = b3.reshape(1, 256)
    w4p = jnp.pad(W4, ((0, 0), (0, 64 - W4.shape[1])))
    b4p = jnp.pad(b4, (0, 64 - b4.shape[0])).reshape(1, 64)

    ones16 = jnp.ones((128, 16), jnp.float32)
    zeros16 = jnp.zeros((RPT, 16), jnp.float32)
    zerosW = jnp.zeros((RPT, W), jnp.float32)

    deg = _make_deg()(src_d, dst_p, ones16, zeros16)
    sv, dv, g0 = _norm_call(deg, x_pad)

    a0 = _make_agg(2)(g0, src_g, dst_p, zerosW)
    g1 = _layer_call(a0, W0, b0r, sv, dv)
    a1 = _make_agg(4)(g1, src_g, dst_p, zerosW)
    g2 = _layer_call(a1, W1, b1r, sv, dv)
    a2 = _make_agg(4)(g2, src_g, dst_p, zerosW)
    g3 = _layer_call(a2, W2, b2r, sv, dv)
    a3 = _make_agg(4)(g3, src_g, dst_p, zerosW)
    z = _tail_call(a3, W3, b3r, w4p, sv, dv)
    za = _make_agg(1)(z, src_g, dst_p, zerosW)
    out = _final_call(za, dv, b4p)
    return out[:N, :40]


# speedup vs baseline: 10.6561x; 1.0371x over previous
import functools

import jax
import jax.numpy as jnp
from jax import lax
from jax.experimental import pallas as pl
from jax.experimental.pallas import tpu as pltpu
from jax.experimental.pallas import tpu_sc as plsc

N = 10000
NP = 10240
E = 320000
EP = 327680
ER = 2560
RPT = NP // 16
NC = 2
NS = 16
W = 64
BN = 1024


@functools.cache
def _mesh():
    return plsc.VectorSubcoreMesh(
        core_axis_name="c", subcore_axis_name="s",
        num_cores=NC, num_subcores=NS)


def _deg_kernel(src_hbm, dst_hbm, ones_hbm, zeros_hbm, out_hbm,
                idx_v, ones_v, deg_sh, ds0, ds1, ds2, ds3):
    dsem = (ds0, ds1, ds2, ds3)
    c = lax.axis_index("c")
    s = lax.axis_index("s")
    cpt = ER // NS

    @pl.when(c == 0)
    def _():
        pltpu.sync_copy(src_hbm.at[pl.ds(s * cpt, cpt)], idx_v)

    @pl.when(c == 1)
    def _():
        pltpu.sync_copy(dst_hbm.at[pl.ds(s * cpt, cpt)], idx_v)

    pltpu.sync_copy(ones_hbm, ones_v)
    pltpu.sync_copy(zeros_hbm, deg_sh.at[pl.ds(s * RPT, RPT)])
    plsc.subcore_barrier()

    def body(g, carry):
        cps = [pltpu.async_copy(ones_v, deg_sh.at[idx_v.at[8 * g + b]],
                                dsem[b % 4], add=True) for b in range(8)]
        for cp in cps:
            cp.wait()
        return carry

    lax.fori_loop(0, cpt // 8, body, 0)
    plsc.subcore_barrier()
    pltpu.sync_copy(deg_sh.at[pl.ds(s * RPT, RPT)],
                    out_hbm.at[c, pl.ds(s * RPT, RPT)])


def _make_deg():
    return functools.partial(
        pl.kernel,
        out_type=jax.ShapeDtypeStruct((NC, NP, 16), jnp.float32),
        mesh=_mesh(),
        compiler_params=pltpu.CompilerParams(use_tc_tiling_on_sc=False),
        scratch_types=[
            pltpu.VMEM((ER // NS, 128), jnp.int32),
            pltpu.VMEM((128, 16), jnp.float32),
            pltpu.VMEM_SHARED((NP, 16), jnp.float32),
            pltpu.SemaphoreType.DMA,
            pltpu.SemaphoreType.DMA,
            pltpu.SemaphoreType.DMA,
            pltpu.SemaphoreType.DMA,
        ],
    )(_deg_kernel)


NB = 5


def _agg_body(nplanes, g_hbm, src_hbm, dst_hbm, zeros_hbm, out_hbm,
              src_v, dst_v, *bufs):
    rows = bufs[:NB]
    agg_sh = bufs[NB]
    gsem = bufs[NB + 1:2 * NB + 1]
    ssem = bufs[2 * NB + 1:3 * NB + 1]
    c = lax.axis_index("c")
    s = lax.axis_index("s")
    if nplanes >= 2:
        cpt = ER // NS
        base = s * cpt
    else:
        cpt = ER // (NC * NS)
        base = (c * NS + s) * cpt
    pltpu.sync_copy(src_hbm.at[pl.ds(base, cpt)], src_v)
    pltpu.sync_copy(dst_hbm.at[pl.ds(base, cpt)], dst_v)

    def one_plane(table, out_ref):
        pltpu.sync_copy(zeros_hbm, agg_sh.at[pl.ds(s * RPT, RPT)])
        plsc.subcore_barrier()

        def gather(ch, b):
            return pltpu.async_copy(table.at[src_v.at[ch]], rows[b], gsem[b])

        def gather_wait(ch, b):
            pltpu.make_async_copy(table.at[src_v.at[ch]], rows[b],
                                  gsem[b]).wait()

        def scatter(ch, b):
            return pltpu.async_copy(rows[b], agg_sh.at[dst_v.at[ch]],
                                    ssem[b], add=True)

        for b in range(NB):
            gather(b, b)

        def body(g, carry):
            cps = []
            for b in range(NB):
                gather_wait(NB * g + b, b)
                cps.append(scatter(NB * g + b, b))
            for b in range(NB):
                cps[b].wait()
                gather(NB * g + NB + b, b)
            return carry

        lax.fori_loop(0, cpt // NB - 1, body, 0)
        cps = []
        for b in range(NB):
            ch = cpt - NB + b
            gather_wait(ch, b)
            cps.append(scatter(ch, b))
        for cp in cps:
            cp.wait()
        plsc.subcore_barrier()
        pltpu.sync_copy(agg_sh.at[pl.ds(s * RPT, RPT)], out_ref)
        plsc.subcore_barrier()

    if nplanes == 1:
        one_plane(g_hbm,
                  out_hbm.at[pl.ds(s * RPT, RPT), pl.ds(64 * c, W)])
    elif nplanes == 2:
        one_plane(g_hbm.at[c],
                  out_hbm.at[pl.ds(s * RPT, RPT), pl.ds(64 * c, W)])
    else:
        for pp in range(nplanes // 2):
            one_plane(g_hbm.at[c * (nplanes // 2) + pp],
                      out_hbm.at[c, pl.ds(s * RPT, RPT), pl.ds(64 * pp, W)])


def _make_agg(nplanes):
    cpt = ER // NS if nplanes >= 2 else ER // (NC * NS)
    out_shape = ((NC, NP, 128) if nplanes == 4 else (NP, 128))
    return functools.partial(
        pl.kernel,
        out_type=jax.ShapeDtypeStruct(out_shape, jnp.float32),
        mesh=_mesh(),
        compiler_params=pltpu.CompilerParams(use_tc_tiling_on_sc=False),
        scratch_types=(
            [pltpu.VMEM((cpt, 128), jnp.int32),
             pltpu.VMEM((cpt, 128), jnp.int32)] +
            [pltpu.VMEM((128, W), jnp.float32) for _ in range(NB)] +
            [pltpu.VMEM_SHARED((NP, W), jnp.float32)] +
            [pltpu.SemaphoreType.DMA for _ in range(2 * NB)]
        ),
    )(functools.partial(_agg_body, nplanes))


def _norm_body(deg_ref, x_ref, s_ref, d_ref, g0_ref):
    s = lax.rsqrt(jnp.maximum(deg_ref[0, :, 0:1], 1.0))
    d = lax.rsqrt(jnp.maximum(deg_ref[1, :, 0:1], 1.0))
    s_ref[...] = s
    d_ref[...] = d
    xs = x_ref[...] * s
    g0_ref[0] = xs[:, 0:W]
    g0_ref[1] = xs[:, W:2 * W]


def _norm_call(deg, x_pad):
    nb = NP // BN
    return pl.pallas_call(
        _norm_body,
        grid=(nb,),
        in_specs=[
            pl.BlockSpec((NC, BN, 16), lambda i: (0, i, 0)),
            pl.BlockSpec((BN, 128), lambda i: (i, 0)),
        ],
        out_specs=(
            pl.BlockSpec((BN, 1), lambda i: (i, 0)),
            pl.BlockSpec((BN, 1), lambda i: (i, 0)),
            pl.BlockSpec((2, BN, W), lambda i: (0, i, 0)),
        ),
        out_shape=(
            jax.ShapeDtypeStruct((NP, 1), jnp.float32),
            jax.ShapeDtypeStruct((NP, 1), jnp.float32),
            jax.ShapeDtypeStruct((2, NP, W), jnp.float32),
        ),
    )(deg, x_pad)


def _layer_body(kin, a_ref, w_ref, b_ref, s_ref, d_ref, out_ref):
    sv = s_ref[...]
    dv = d_ref[...]
    w = w_ref[...]
    for q in (0, 1):
        acc = None
        for k in range(kin):
            a = a_ref[...] if kin == 1 else a_ref[k]
            t = jnp.dot(a, w[k * 128:(k + 1) * 128, q * 128:(q + 1) * 128],
                        preferred_element_type=jnp.float32)
            acc = t if acc is None else acc + t
        h = sv * jnp.maximum(dv * acc + b_ref[0:1, q * 128:(q + 1) * 128],
                             0.0)
        out_ref[2 * q] = h[:, 0:W]
        out_ref[2 * q + 1] = h[:, W:2 * W]


def _layer_call(a, w, b, sv, dv):
    nb = NP // BN
    kin = 1 if a.ndim == 2 else a.shape[0]
    a_spec = (pl.BlockSpec((BN, 128), lambda i: (i, 0)) if kin == 1
              else pl.BlockSpec((kin, BN, 128), lambda i: (0, i, 0)))
    kdim = w.shape[0]
    return pl.pallas_call(
        functools.partial(_layer_body, kin),
        grid=(nb,),
        in_specs=[
            a_spec,
            pl.BlockSpec((kdim, 256), lambda i: (0, 0)),
            pl.BlockSpec((1, 256), lambda i: (0, 0)),
            pl.BlockSpec((BN, 1), lambda i: (i, 0)),
            pl.BlockSpec((BN, 1), lambda i: (i, 0)),
        ],
        out_specs=pl.BlockSpec((4, BN, W), lambda i: (0, i, 0)),
        out_shape=jax.ShapeDtypeStruct((4, NP, W), jnp.float32),
    )(a, w, b, sv, dv)


def _tail_body(a_ref, w3_ref, b3_ref, w4_ref, s_ref, d_ref, z_ref):
    sv = s_ref[...]
    dv = d_ref[...]
    w3 = w3_ref[...]
    z = None
    for q in (0, 1):
        acc = None
        for k in range(2):
            t = jnp.dot(a_ref[k],
                        w3[k * 128:(k + 1) * 128, q * 128:(q + 1) * 128],
                        preferred_element_type=jnp.float32)
            acc = t if acc is None else acc + t
        h = sv * jnp.maximum(dv * acc + b3_ref[0:1, q * 128:(q + 1) * 128],
                             0.0)
        zq = jnp.dot(h, w4_ref[q * 128:(q + 1) * 128, :],
                     preferred_element_type=jnp.float32)
        z = zq if z is None else z + zq
    z_ref[...] = z


def _tail_call(a, w3, b3, w4p, sv, dv):
    nb = NP // BN
    return pl.pallas_call(
        _tail_body,
        grid=(nb,),
        in_specs=[
            pl.BlockSpec((NC, BN, 128), lambda i: (0, i, 0)),
            pl.BlockSpec((256, 256), lambda i: (0, 0)),
            pl.BlockSpec((1, 256), lambda i: (0, 0)),
            pl.BlockSpec((256, 64), lambda i: (0, 0)),
            pl.BlockSpec((BN, 1), lambda i: (i, 0)),
            pl.BlockSpec((BN, 1), lambda i: (i, 0)),
        ],
        out_specs=pl.BlockSpec((BN, 64), lambda i: (i, 0)),
        out_shape=jax.ShapeDtypeStruct((NP, 64), jnp.float32),
    )(a, w3, b3, w4p, sv, dv)


def _final_body(z_ref, d_ref, b4_ref, out_ref):
    z = z_ref[...]
    out_ref[...] = d_ref[...] * (z[:, 0:W] + z[:, W:2 * W]) + b4_ref[...]


def _final_call(z, dv, b4p):
    nb = NP // BN
    return pl.pallas_call(
        _final_body,
        grid=(nb,),
        in_specs=[
            pl.BlockSpec((BN, 128), lambda i: (i, 0)),
            pl.BlockSpec((BN, 1), lambda i: (i, 0)),
            pl.BlockSpec((1, 64), lambda i: (0, 0)),
        ],
        out_specs=pl.BlockSpec((BN, 64), lambda i: (i, 0)),
        out_shape=jax.ShapeDtypeStruct((NP, 64), jnp.float32),
    )(z, dv, b4p)


def kernel(x, edge_index, W0, b0, W1, b1, W2, b2, W3, b3, W4, b4):
    src = edge_index[0].astype(jnp.int32)
    dst = edge_index[1].astype(jnp.int32)
    pad = EP - E
    sweep = jnp.arange(pad, dtype=jnp.int32)
    pad_gather = sweep * 67 % N
    pad_dump = N + sweep % (NP - N)
    src_g = jnp.concatenate([src, pad_gather]).reshape(ER, 128)
    src_d = jnp.concatenate([src, pad_dump]).reshape(ER, 128)
    dst_p = jnp.concatenate([dst, pad_dump]).reshape(ER, 128)

    x_pad = jnp.pad(x, ((0, NP - N), (0, 0)))
    b0r = b0.reshape(1, 256)
    b1r = b1.reshape(1, 256)
    b2r = b2.reshape(1, 256)
    b3r = b3.reshape(1, 256)
    w4p = jnp.pad(W4, ((0, 0), (0, 64 - W4.shape[1])))
    b4p = jnp.pad(b4, (0, 64 - b4.shape[0])).reshape(1, 64)

    ones16 = jnp.ones((128, 16), jnp.float32)
    zeros16 = jnp.zeros((RPT, 16), jnp.float32)
    zerosW = jnp.zeros((RPT, W), jnp.float32)

    deg = _make_deg()(src_d, dst_p, ones16, zeros16)
    sv, dv, g0 = _norm_call(deg, x_pad)

    a0 = _make_agg(2)(g0, src_g, dst_p, zerosW)
    g1 = _layer_call(a0, W0, b0r, sv, dv)
    a1 = _make_agg(4)(g1, src_g, dst_p, zerosW)
    g2 = _layer_call(a1, W1, b1r, sv, dv)
    a2 = _make_agg(4)(g2, src_g, dst_p, zerosW)
    g3 = _layer_call(a2, W2, b2r, sv, dv)
    a3 = _make_agg(4)(g3, src_g, dst_p, zerosW)
    z = _tail_call(a3, W3, b3r, w4p, sv, dv)
    za = _make_agg(1)(z, src_g, dst_p, zerosW)
    out = _final_call(za, dv, b4p)
    return out[:N, :40]
